# baseline (speedup 1.0000x reference)
import functools

import jax
import jax.numpy as jnp
from jax.experimental import pallas as pl
from jax.experimental.pallas import tpu as pltpu

N = 10000
D_IN = 128
D_OUT = 128
CHUNK_ROWS = 400
NBUF = 3
NCHUNKS = N // CHUNK_ROWS
NPIECE = 5
PIECE_ROWS = CHUNK_ROWS // NPIECE


def _gcn_kernel(x_ref, a_ref, w_ref, o_ref, s_ref, buf_ref, sem_ref, tsem_ref):
    i = pl.program_id(0)
    last = NCHUNKS - 1

    def chunk_copy(chunk_idx, slot):
        return pltpu.make_async_copy(
            a_ref.at[pl.ds(chunk_idx * CHUNK_ROWS, CHUNK_ROWS), :],
            buf_ref.at[slot],
            sem_ref.at[slot],
        )

    def piece_copy(piece_idx, slot):
        return pltpu.make_async_copy(
            a_ref.at[pl.ds(last * CHUNK_ROWS + piece_idx * PIECE_ROWS,
                           PIECE_ROWS), :],
            buf_ref.at[slot, pl.ds(piece_idx * PIECE_ROWS, PIECE_ROWS), :],
            tsem_ref.at[piece_idx],
        )

    @pl.when(i == 0)
    def _bootstrap():
        for slot in range(min(NBUF, NCHUNKS - 1)):
            chunk_copy(slot, slot).start()
        s_ref[...] = jnp.dot(
            x_ref[...], w_ref[...], preferred_element_type=jnp.float32
        )

    slot = jax.lax.rem(i, NBUF)

    @pl.when(i < last)
    def _steady():
        chunk_copy(i, slot).wait()
        o_ref[...] = jnp.dot(
            buf_ref[slot], s_ref[...], preferred_element_type=jnp.float32
        )

    @pl.when(i + NBUF < last)
    def _prefetch_full():
        chunk_copy(i + NBUF, slot).start()

    @pl.when(i + NBUF == last)
    def _prefetch_tail_pieces():
        for p in range(NPIECE):
            piece_copy(p, jax.lax.rem(jnp.int32(last), NBUF)).start()

    @pl.when(i == last)
    def _tail():
        for p in range(NPIECE):
            piece_copy(p, slot).wait()
            rows = pl.ds(p * PIECE_ROWS, PIECE_ROWS)
            o_ref[rows, :] = jnp.dot(
                buf_ref[slot, rows, :], s_ref[...],
                preferred_element_type=jnp.float32,
            )


@functools.partial(jax.jit, static_argnames=())
def kernel(X, A, W):
    n, d_in = X.shape
    d_out = W.shape[1]
    return pl.pallas_call(
        _gcn_kernel,
        grid=(NCHUNKS,),
        in_specs=[
            pl.BlockSpec((n, d_in), lambda i: (0, 0)),
            pl.BlockSpec(memory_space=pltpu.MemorySpace.HBM),
            pl.BlockSpec((d_in, d_out), lambda i: (0, 0)),
        ],
        out_specs=pl.BlockSpec((CHUNK_ROWS, d_out), lambda i: (i, 0)),
        out_shape=jax.ShapeDtypeStruct((n, d_out), jnp.float32),
        scratch_shapes=[
            pltpu.VMEM((n, d_out), jnp.float32),
            pltpu.VMEM((NBUF, CHUNK_ROWS, n), jnp.float32),
            pltpu.SemaphoreType.DMA((NBUF,)),
            pltpu.SemaphoreType.DMA((NPIECE,)),
        ],
        compiler_params=pltpu.CompilerParams(
            vmem_limit_bytes=120 * 1024 * 1024,
        ),
    )(X, A, W)


# speedup vs baseline: 1.0287x; 1.0287x over previous
import functools

import jax
import jax.numpy as jnp
from jax.experimental import pallas as pl
from jax.experimental.pallas import tpu as pltpu

N = 10000
D_IN = 128
D_OUT = 128
BLOCK_ROWS = 400


def _gcn_kernel(x_ref, a_ref, w_ref, o_ref, s_ref):
    @pl.when(pl.program_id(0) == 0)
    def _compute_support():
        s_ref[...] = jnp.dot(
            x_ref[...], w_ref[...], preferred_element_type=jnp.float32
        ).astype(jnp.bfloat16)

    o_ref[...] = jnp.dot(
        a_ref[...].astype(jnp.bfloat16),
        s_ref[...],
        preferred_element_type=jnp.float32,
    )


@functools.partial(jax.jit, static_argnames=())
def kernel(X, A, W):
    n, d_in = X.shape
    d_out = W.shape[1]
    grid = (pl.cdiv(n, BLOCK_ROWS),)
    return pl.pallas_call(
        _gcn_kernel,
        grid=grid,
        in_specs=[
            pl.BlockSpec((n, d_in), lambda i: (0, 0)),
            pl.BlockSpec((BLOCK_ROWS, n), lambda i: (i, 0)),
            pl.BlockSpec((d_in, d_out), lambda i: (0, 0)),
        ],
        out_specs=pl.BlockSpec((BLOCK_ROWS, d_out), lambda i: (i, 0)),
        out_shape=jax.ShapeDtypeStruct((n, d_out), jnp.float32),
        scratch_shapes=[pltpu.VMEM((n, d_out), jnp.bfloat16)],
        compiler_params=pltpu.CompilerParams(
            vmem_limit_bytes=120 * 1024 * 1024,
        ),
    )(X, A, W)
